# baseline (speedup 1.0000x reference)
import functools

import jax
import jax.numpy as jnp
from jax import lax
from jax.experimental import pallas as pl
from jax.experimental.pallas import tpu as pltpu
from jax.experimental.pallas import tpu_sc as plsc

_B = 16384
_SEQ = 50
_E = 32
_H = 128
_NCLS = 1000

_NC = 2
_NS = 16
_NW = _NC * _NS
_RPT = _B // _NW
_CB = 8
_NCH = _RPT // _CB
_PADW = 64
_SENT = -1
_EP = 40


def _sc_body(idx_hbm, table_hbm, out_hbm,
             idx0, idx1, rows0, rows1, outc0, outc1,
             isem0, isem1, gsem0, gsem1, osem0, osem1):
    wid = lax.axis_index("s") * _NC + lax.axis_index("c")
    base = wid * _RPT
    idxc = (idx0, idx1)
    rows = (rows0, rows1)
    outc = (outc0, outc1)
    isems = (isem0, isem1)
    gsems = (gsem0, gsem1)
    osems = (osem0, osem1)

    def start_idx(c, b):
        pltpu.async_copy(
            idx_hbm.at[pl.ds(base + c * _CB, _CB)], idxc[b], isems[b])

    def start_gather(c, b):
        pltpu.make_async_copy(
            idx_hbm.at[pl.ds(base + c * _CB, _CB)], idxc[b], isems[b]).wait()
        for r in range(_CB):
            pltpu.async_copy(
                table_hbm.at[plsc.Indices(idxc[b].at[r], ignored_value=_SENT)],
                rows[b].at[r], gsems[b])

    def drain_gather(c, b):
        for r in range(_CB):
            pltpu.make_async_copy(
                table_hbm.at[plsc.Indices(idxc[b].at[r], ignored_value=_SENT)],
                rows[b].at[r], gsems[b]).wait()

    start_idx(0, 0)
    start_idx(1, 1)
    start_gather(0, 0)
    start_gather(1, 1)

    def outer(c2, carry):
        for b in range(2):
            c = c2 * 2 + b
            drain_gather(c, b)
            @pl.when(c + 2 < _NCH)
            def _():
                start_idx(c + 2, b)

            @pl.when(c >= 2)
            def _():
                pltpu.make_async_copy(
                    outc[b], out_hbm.at[pl.ds(base + (c - 2) * _CB, _CB)],
                    osems[b]).wait()

            def row_body(r, carry2):
                a0 = rows[b][r, 0, pl.ds(0, 16)]
                a1 = rows[b][r, 0, pl.ds(16, 16)]
                for j in range(1, _SEQ):
                    a0 = a0 + rows[b][r, j, pl.ds(0, 16)]
                    a1 = a1 + rows[b][r, j, pl.ds(16, 16)]
                outc[b][r, pl.ds(0, 16)] = a0
                outc[b][r, pl.ds(16, 16)] = a1
                return carry2

            lax.fori_loop(0, _CB, row_body, 0)
            pltpu.async_copy(
                outc[b], out_hbm.at[pl.ds(base + c * _CB, _CB)], osems[b])

            @pl.when(c + 2 < _NCH)
            def _():
                start_gather(c + 2, b)
        return carry

    lax.fori_loop(0, _NCH // 2, outer, 0)
    for b in range(2):
        c = _NCH - 2 + b
        pltpu.make_async_copy(
            outc[b], out_hbm.at[pl.ds(base + c * _CB, _CB)], osems[b]).wait()


_sc_gather_sum = functools.partial(
    pl.kernel,
    out_type=jax.ShapeDtypeStruct((_B, _E), jnp.float32),
    mesh=plsc.VectorSubcoreMesh(
        core_axis_name="c", subcore_axis_name="s",
        num_cores=_NC, num_subcores=_NS),
    scratch_types=[
        pltpu.VMEM((_CB, _PADW), jnp.int32),
        pltpu.VMEM((_CB, _PADW), jnp.int32),
        pltpu.VMEM((_CB, _PADW, _EP), jnp.float32),
        pltpu.VMEM((_CB, _PADW, _EP), jnp.float32),
        pltpu.VMEM((_CB, _E), jnp.float32),
        pltpu.VMEM((_CB, _E), jnp.float32),
        pltpu.SemaphoreType.DMA,
        pltpu.SemaphoreType.DMA,
        pltpu.SemaphoreType.DMA,
        pltpu.SemaphoreType.DMA,
        pltpu.SemaphoreType.DMA,
        pltpu.SemaphoreType.DMA,
    ],
    compiler_params=pltpu.CompilerParams(use_tc_tiling_on_sc=False),
)(_sc_body)


_BB = 512


def _mlp_body(x_ref, w1_ref, b1_ref, w2_ref, b2_ref, w3_ref, b3_ref, o_ref):
    x = x_ref[...]
    h = jnp.dot(x, w1_ref[...], preferred_element_type=jnp.float32) + b1_ref[...]
    h = jnp.maximum(h, 0.0)
    h = jnp.dot(h, w2_ref[...], preferred_element_type=jnp.float32) + b2_ref[...]
    h = jnp.maximum(h, 0.0)
    o = jnp.dot(h, w3_ref[...], preferred_element_type=jnp.float32) + b3_ref[...]
    m = jnp.max(o, axis=-1, keepdims=True)
    e = jnp.exp(o - m)
    s = jnp.log(jnp.sum(e, axis=-1, keepdims=True))
    o_ref[...] = jnp.swapaxes(o - m - s, 0, 1)


def _mlp(embeds, W1, b1, W2, b2, W3, b3):
    grid = (_B // _BB,)
    return pl.pallas_call(
        _mlp_body,
        grid=grid,
        in_specs=[
            pl.BlockSpec((_BB, _E), lambda i: (i, 0)),
            pl.BlockSpec((_E, _H), lambda i: (0, 0)),
            pl.BlockSpec((1, _H), lambda i: (0, 0)),
            pl.BlockSpec((_H, _H), lambda i: (0, 0)),
            pl.BlockSpec((1, _H), lambda i: (0, 0)),
            pl.BlockSpec((_H, _NCLS), lambda i: (0, 0)),
            pl.BlockSpec((1, _NCLS), lambda i: (0, 0)),
        ],
        out_specs=pl.BlockSpec((_NCLS, _BB), lambda i: (0, i)),
        out_shape=jax.ShapeDtypeStruct((_NCLS, _B), jnp.float32),
    )(embeds, W1, b1.reshape(1, _H), W2, b2.reshape(1, _H),
      W3, b3.reshape(1, _NCLS)).T


def kernel(indices, table, W1, b1, W2, b2, W3, b3):
    idx = jnp.pad(indices.astype(jnp.int32), ((0, 0), (0, _PADW - _SEQ)),
                  constant_values=_SENT)
    tbl = jnp.pad(table, ((0, 0), (0, _EP - _E)))
    embeds = _sc_gather_sum(idx, tbl)
    return _mlp(embeds, W1, b1, W2, b2, W3, b3)


# speedup vs baseline: 1.7583x; 1.7583x over previous
import functools

import jax
import jax.numpy as jnp
from jax import lax
from jax.experimental import pallas as pl
from jax.experimental.pallas import tpu as pltpu
from jax.experimental.pallas import tpu_sc as plsc

_B = 16384
_SEQ = 50
_E = 32
_H = 128
_NCLS = 1000

_NC = 2
_NS = 16
_NW = _NC * _NS
_RPT = _B // _NW
_CB = 16
_NCH = _RPT // _CB
_PADW = 64
_SENT = -1


def _sc_body(idx_hbm, table_hbm, out_hbm,
             idx0, idx1, rows0, rows1, outc0, outc1,
             isem0, isem1, gsem0, gsem1, osem0, osem1):
    wid = lax.axis_index("s") * _NC + lax.axis_index("c")
    base = wid * _RPT
    idxc = (idx0, idx1)
    rows = (rows0, rows1)
    outc = (outc0, outc1)
    isems = (isem0, isem1)
    gsems = (gsem0, gsem1)
    osems = (osem0, osem1)

    def start_idx(c, b):
        pltpu.async_copy(
            idx_hbm.at[pl.ds(base + c * _CB, _CB)], idxc[b], isems[b])

    def start_gather(c, b):
        pltpu.make_async_copy(
            idx_hbm.at[pl.ds(base + c * _CB, _CB)], idxc[b], isems[b]).wait()
        for r in range(_CB):
            pltpu.async_copy(
                table_hbm.at[plsc.Indices(idxc[b].at[r], ignored_value=_SENT)],
                rows[b].at[r], gsems[b])

    def drain_gather(c, b):
        for r in range(_CB):
            pltpu.make_async_copy(
                table_hbm.at[plsc.Indices(idxc[b].at[r], ignored_value=_SENT)],
                rows[b].at[r], gsems[b]).wait()

    start_idx(0, 0)
    start_idx(1, 1)
    start_gather(0, 0)
    start_gather(1, 1)

    def outer(c2, carry):
        for b in range(2):
            c = c2 * 2 + b
            drain_gather(c, b)
            @pl.when(c + 2 < _NCH)
            def _():
                start_idx(c + 2, b)

            @pl.when(c >= 2)
            def _():
                pltpu.make_async_copy(
                    outc[b], out_hbm.at[pl.ds(base + (c - 2) * _CB, _CB)],
                    osems[b]).wait()

            def row_body(r, carry2):
                a0 = rows[b][r, 0, pl.ds(0, 16)]
                a1 = rows[b][r, 0, pl.ds(16, 16)]
                for j in range(1, _SEQ):
                    a0 = a0 + rows[b][r, j, pl.ds(0, 16)]
                    a1 = a1 + rows[b][r, j, pl.ds(16, 16)]
                outc[b][r, pl.ds(0, 16)] = a0
                outc[b][r, pl.ds(16, 16)] = a1
                return carry2

            lax.fori_loop(0, _CB, row_body, 0)
            pltpu.async_copy(
                outc[b], out_hbm.at[pl.ds(base + c * _CB, _CB)], osems[b])

            @pl.when(c + 2 < _NCH)
            def _():
                start_gather(c + 2, b)
        return carry

    lax.fori_loop(0, _NCH // 2, outer, 0)
    for b in range(2):
        c = _NCH - 2 + b
        pltpu.make_async_copy(
            outc[b], out_hbm.at[pl.ds(base + c * _CB, _CB)], osems[b]).wait()


_sc_gather_sum = functools.partial(
    pl.kernel,
    out_type=jax.ShapeDtypeStruct((_B, _E), jnp.float32),
    mesh=plsc.VectorSubcoreMesh(
        core_axis_name="c", subcore_axis_name="s",
        num_cores=_NC, num_subcores=_NS),
    scratch_types=[
        pltpu.VMEM((_CB, _PADW), jnp.int32),
        pltpu.VMEM((_CB, _PADW), jnp.int32),
        pltpu.VMEM((_CB, _PADW, _E), jnp.float32),
        pltpu.VMEM((_CB, _PADW, _E), jnp.float32),
        pltpu.VMEM((_CB, _E), jnp.float32),
        pltpu.VMEM((_CB, _E), jnp.float32),
        pltpu.SemaphoreType.DMA,
        pltpu.SemaphoreType.DMA,
        pltpu.SemaphoreType.DMA,
        pltpu.SemaphoreType.DMA,
        pltpu.SemaphoreType.DMA,
        pltpu.SemaphoreType.DMA,
    ],
    compiler_params=pltpu.CompilerParams(use_tc_tiling_on_sc=False),
)(_sc_body)


_BB = 512


def _mlp_body(x_ref, w1_ref, b1_ref, w2_ref, b2_ref, w3_ref, b3_ref, o_ref):
    x = x_ref[...]
    h = jnp.dot(x, w1_ref[...], preferred_element_type=jnp.float32) + b1_ref[...]
    h = jnp.maximum(h, 0.0)
    h = jnp.dot(h, w2_ref[...], preferred_element_type=jnp.float32) + b2_ref[...]
    h = jnp.maximum(h, 0.0)
    o = jnp.dot(h, w3_ref[...], preferred_element_type=jnp.float32) + b3_ref[...]
    m = jnp.max(o, axis=-1, keepdims=True)
    e = jnp.exp(o - m)
    s = jnp.log(jnp.sum(e, axis=-1, keepdims=True))
    o_ref[...] = jnp.swapaxes(o - m - s, 0, 1)


def _mlp(embeds, W1, b1, W2, b2, W3, b3):
    grid = (_B // _BB,)
    return pl.pallas_call(
        _mlp_body,
        grid=grid,
        in_specs=[
            pl.BlockSpec((_BB, _E), lambda i: (i, 0)),
            pl.BlockSpec((_E, _H), lambda i: (0, 0)),
            pl.BlockSpec((1, _H), lambda i: (0, 0)),
            pl.BlockSpec((_H, _H), lambda i: (0, 0)),
            pl.BlockSpec((1, _H), lambda i: (0, 0)),
            pl.BlockSpec((_H, _NCLS), lambda i: (0, 0)),
            pl.BlockSpec((1, _NCLS), lambda i: (0, 0)),
        ],
        out_specs=pl.BlockSpec((_NCLS, _BB), lambda i: (0, i)),
        out_shape=jax.ShapeDtypeStruct((_NCLS, _B), jnp.float32),
    )(embeds, W1, b1.reshape(1, _H), W2, b2.reshape(1, _H),
      W3, b3.reshape(1, _NCLS)).T


def kernel(indices, table, W1, b1, W2, b2, W3, b3):
    idx = jnp.pad(indices.astype(jnp.int32), ((0, 0), (0, _PADW - _SEQ)),
                  constant_values=_SENT)
    embeds = _sc_gather_sum(idx, table)
    return _mlp(embeds, W1, b1, W2, b2, W3, b3)
